# baseline (speedup 1.0000x reference)
import functools

import jax
import jax.numpy as jnp
from jax import lax
from jax.experimental import pallas as pl
from jax.experimental.pallas import tpu as pltpu
from jax.experimental.pallas import tpu_sc as plsc

NUM_EMBEDDINGS = 1000000
EMBEDDING_DIM = 32
BATCH = 16384
HIST_LEN = 50

_INFO = plsc.get_sparse_core_info()
_NC = _INFO.num_cores
_NS = _INFO.num_subcores
_NW = _NC * _NS

_BT = 128
_NBT = BATCH // _BT
_BT_PER_W = _NBT // _NW
_C_PER_W = _BT_PER_W * HIST_LEN
_RING = 4
_DT = EMBEDDING_DIM // 8

assert _C_PER_W % _RING == 0


def _sc_gather(ids_t, table):
    mesh = plsc.VectorSubcoreMesh(core_axis_name="c", subcore_axis_name="s")

    @functools.partial(
        pl.kernel,
        mesh=mesh,
        out_type=jax.ShapeDtypeStruct((HIST_LEN, _DT, _NBT, 8, _BT), jnp.float32),
        scratch_types=[
            pltpu.VMEM((HIST_LEN, _BT_PER_W * _BT), jnp.int32),
            pltpu.VMEM((_RING, _BT, EMBEDDING_DIM), jnp.float32),
            pltpu.VMEM((_RING, _DT, 8, _BT), jnp.float32),
            pltpu.SemaphoreType.DMA,
            pltpu.SemaphoreType.DMA,
            pltpu.SemaphoreType.DMA,
            pltpu.SemaphoreType.DMA,
            pltpu.SemaphoreType.DMA,
            pltpu.SemaphoreType.DMA,
            pltpu.SemaphoreType.DMA,
            pltpu.SemaphoreType.DMA,
        ],
        compiler_params=pltpu.CompilerParams(
            use_tc_tiling_on_sc=False, needs_layout_passes=False),
    )
    def k(ids_hbm, table_hbm, out_hbm, idx_v, rows_v, sbuf_v,
          si0, si1, si2, si3, so0, so1, so2, so3):
        sis = (si0, si1, si2, si3)
        sos = (so0, so1, so2, so3)
        wid = lax.axis_index("s") * _NC + lax.axis_index("c")
        bt0 = wid * _BT_PER_W

        pltpu.sync_copy(
            ids_hbm.at[:, pl.ds(bt0 * _BT, _BT_PER_W * _BT)], idx_v)

        def fire_gather(t, r):
            h = t % HIST_LEN
            bo = t // HIST_LEN
            pltpu.async_copy(
                table_hbm.at[idx_v.at[h, pl.ds(bo * _BT, _BT)]],
                rows_v.at[r], sis[r])

        def wait_gather(r):
            pltpu.make_async_copy(
                table_hbm.at[pl.ds(0, _BT)], rows_v.at[r], sis[r]).wait()

        def transpose(r):
            @plsc.parallel_loop(0, _BT // 16, unroll=4)
            def g_body(g):
                base = g * 16
                lanes = base + lax.iota(jnp.int32, 16)
                for d in range(EMBEDDING_DIM):
                    v = plsc.load_gather(
                        rows_v.at[r],
                        [lanes, jnp.full((16,), d, jnp.int32)])
                    sbuf_v[r, d // 8, d % 8, pl.ds(base, 16)] = v

        def fire_stores(t, r):
            h = t % HIST_LEN
            bo = t // HIST_LEN
            for dt in range(_DT):
                pltpu.async_copy(
                    sbuf_v.at[r, dt], out_hbm.at[h, dt, bt0 + bo], sos[r])

        def wait_stores(r):
            for dt in range(_DT):
                pltpu.make_async_copy(
                    sbuf_v.at[r, dt], out_hbm.at[0, dt, 0], sos[r]).wait()

        for r in range(_RING):
            fire_gather(r, r)

        def do_round(i, first, last):
            for r in range(_RING):
                t = i * _RING + r
                wait_gather(r)
                if not first:
                    wait_stores(r)
                transpose(r)
                fire_stores(t, r)
                if not last:
                    fire_gather(t + _RING, r)

        n_rounds = _C_PER_W // _RING
        do_round(0, True, False)

        def body(i, carry):
            do_round(i, False, False)
            return carry

        lax.fori_loop(1, n_rounds - 1, body, 0)
        do_round(n_rounds - 1, False, True)
        for r in range(_RING):
            wait_stores(r)

    return k(ids_t, table)


def kernel(input_ids, table):
    ids_t = input_ids.T.astype(jnp.int32)
    y = _sc_gather(ids_t, table)
    return y.transpose(2, 4, 0, 1, 3).reshape(BATCH, HIST_LEN, EMBEDDING_DIM)


# speedup vs baseline: 1.1442x; 1.1442x over previous
import functools

import jax
import jax.numpy as jnp
from jax import lax
from jax.experimental import pallas as pl
from jax.experimental.pallas import tpu as pltpu
from jax.experimental.pallas import tpu_sc as plsc

NUM_EMBEDDINGS = 1000000
EMBEDDING_DIM = 32
BATCH = 16384
HIST_LEN = 50

_INFO = plsc.get_sparse_core_info()
_NC = _INFO.num_cores
_NS = _INFO.num_subcores
_NW = _NC * _NS

_BT = 128
_NBT = BATCH // _BT
_BT_PER_W = _NBT // _NW
_C_PER_W = _BT_PER_W * HIST_LEN
_RING = 4
_DT = EMBEDDING_DIM // 8

assert _C_PER_W % _RING == 0


def _sc_gather(ids_t, table):
    mesh = plsc.VectorSubcoreMesh(core_axis_name="c", subcore_axis_name="s")

    @functools.partial(
        pl.kernel,
        mesh=mesh,
        out_type=jax.ShapeDtypeStruct((HIST_LEN, _DT, _NBT, 8, _BT), jnp.float32),
        scratch_types=[
            pltpu.VMEM((HIST_LEN, _BT_PER_W * _BT), jnp.int32),
            pltpu.VMEM((_RING, _BT, EMBEDDING_DIM), jnp.float32),
            pltpu.VMEM((_RING, EMBEDDING_DIM, _BT), jnp.float32),
            pltpu.SemaphoreType.DMA,
            pltpu.SemaphoreType.DMA,
            pltpu.SemaphoreType.DMA,
            pltpu.SemaphoreType.DMA,
            pltpu.SemaphoreType.DMA,
            pltpu.SemaphoreType.DMA,
            pltpu.SemaphoreType.DMA,
            pltpu.SemaphoreType.DMA,
        ],
        compiler_params=pltpu.CompilerParams(
            use_tc_tiling_on_sc=False, needs_layout_passes=False),
    )
    def k(ids_hbm, table_hbm, out_hbm, idx_v, rows_v, sbuf_v,
          si0, si1, si2, si3, so0, so1, so2, so3):
        sis = (si0, si1, si2, si3)
        sos = (so0, so1, so2, so3)
        wid = lax.axis_index("s") * _NC + lax.axis_index("c")
        bt0 = wid * _BT_PER_W

        pltpu.sync_copy(
            ids_hbm.at[:, pl.ds(bt0 * _BT, _BT_PER_W * _BT)], idx_v)

        def fire_gather(t, r):
            h = t % HIST_LEN
            bo = t // HIST_LEN
            pltpu.async_copy(
                table_hbm.at[idx_v.at[h, pl.ds(bo * _BT, _BT)]],
                rows_v.at[r], sis[r])

        def wait_gather(r):
            pltpu.make_async_copy(
                table_hbm.at[pl.ds(0, _BT)], rows_v.at[r], sis[r]).wait()

        def transpose(r):
            @plsc.parallel_loop(0, EMBEDDING_DIM, unroll=4)
            def d_body(d):
                dv = jnp.full((16,), d, jnp.int32)
                for g in range(_BT // 16):
                    v = plsc.load_gather(
                        rows_v.at[r],
                        [g * 16 + lax.iota(jnp.int32, 16), dv])
                    sbuf_v[r, d, pl.ds(g * 16, 16)] = v

        def fire_stores(t, r):
            h = t % HIST_LEN
            bo = t // HIST_LEN
            for dt in range(_DT):
                pltpu.async_copy(
                    sbuf_v.at[r, pl.ds(dt * 8, 8)],
                    out_hbm.at[h, dt, bt0 + bo], sos[r])

        def wait_stores(r):
            for dt in range(_DT):
                pltpu.make_async_copy(
                    sbuf_v.at[r, pl.ds(dt * 8, 8)],
                    out_hbm.at[0, dt, 0], sos[r]).wait()

        for r in range(_RING):
            fire_gather(r, r)

        def do_round(i, first, last):
            for r in range(_RING):
                t = i * _RING + r
                wait_gather(r)
                if not first:
                    wait_stores(r)
                transpose(r)
                fire_stores(t, r)
                if not last:
                    fire_gather(t + _RING, r)

        n_rounds = _C_PER_W // _RING
        do_round(0, True, False)

        def body(i, carry):
            do_round(i, False, False)
            return carry

        lax.fori_loop(1, n_rounds - 1, body, 0)
        do_round(n_rounds - 1, False, True)
        for r in range(_RING):
            wait_stores(r)

    return k(ids_t, table)


def kernel(input_ids, table):
    ids_t = input_ids.T.astype(jnp.int32)
    y = _sc_gather(ids_t, table)
    return y.transpose(2, 4, 0, 1, 3).reshape(BATCH, HIST_LEN, EMBEDDING_DIM)
